# baseline (speedup 1.0000x reference)
import jax
import jax.numpy as jnp
from jax import lax
from jax.experimental import pallas as pl
from jax.experimental.pallas import tpu as pltpu
from jax.experimental.pallas import tpu_sc as plsc

VOCAB = 1000
VPAD = 1024
BT = 4096 * 50
NC, NS, L = 2, 16, 16
NW = NC * NS
PER_W = BT // NW
CHUNK = 32
NCHUNK = PER_W // CHUNK
NBUF = 2


def _logz_body(t_ref, o_ref):
    x = t_ref[...]
    m = jnp.max(x, axis=1)
    s = jnp.sum(jnp.exp(x - m[:, None]), axis=1)
    o_ref[...] = m + jnp.log(s)


def _row_logsumexp(table):
    return pl.pallas_call(
        _logz_body,
        out_shape=jax.ShapeDtypeStruct((VOCAB,), jnp.float32),
    )(table)


def _gather_body(tablep_hbm, idx_hbm, outp_hbm,
                 idx_v, rows0_v, rows1_v, sem0, sem1):
    wid = lax.axis_index("s") * NC + lax.axis_index("c")
    base = wid * PER_W
    rows_bufs = (rows0_v, rows1_v)
    sems = (sem0, sem1)

    pltpu.sync_copy(idx_hbm.at[pl.ds(base, PER_W)], idx_v)

    def gather(c, b):
        return pltpu.async_copy(
            tablep_hbm.at[idx_v.at[pl.ds(c * CHUNK, CHUNK)]],
            rows_bufs[b], sems[b])

    for b in range(NBUF):
        gather(b, b)

    def outer_step(g, carry):
        for b in range(NBUF):
            c = g * NBUF + b
            rows_v = rows_bufs[b]
            pltpu.make_async_copy(
                tablep_hbm.at[idx_v.at[pl.ds(c * CHUNK, CHUNK)]],
                rows_v, sems[b]).wait()
            pltpu.sync_copy(rows_v, outp_hbm.at[pl.ds(base + c * CHUNK, CHUNK)])

            @pl.when(c + NBUF < NCHUNK)
            def _():
                gather(c + NBUF, b)
        return carry

    lax.fori_loop(0, NCHUNK // NBUF, outer_step, 0)


def _sc_gather(tablep, idx_flat, width):
    mesh = plsc.VectorSubcoreMesh(core_axis_name="c", subcore_axis_name="s")
    f = pl.kernel(
        _gather_body,
        out_type=jax.ShapeDtypeStruct((BT, width), jnp.float32),
        mesh=mesh,
        compiler_params=pltpu.CompilerParams(
            needs_layout_passes=False, use_tc_tiling_on_sc=False),
        scratch_types=[
            pltpu.VMEM((PER_W,), jnp.int32),
            pltpu.VMEM((CHUNK, width), jnp.float32),
            pltpu.VMEM((CHUNK, width), jnp.float32),
            pltpu.SemaphoreType.DMA,
            pltpu.SemaphoreType.DMA,
        ],
    )
    return f(tablep, idx_flat)


GRP = 128
NGRP = PER_W // GRP


def _loss_body(tflat_hbm, idx_hbm, tgt_hbm, logz_hbm, part_hbm,
               idx_v, tgt_v, fi_v, picked_v, logz_v, acc_v, sem):
    wid = lax.axis_index("s") * NC + lax.axis_index("c")
    base = wid * PER_W

    pltpu.sync_copy(idx_hbm.at[pl.ds(base, PER_W)], idx_v)
    pltpu.sync_copy(tgt_hbm.at[pl.ds(base, PER_W)], tgt_v)
    pltpu.sync_copy(logz_hbm, logz_v)

    def fi_step(g, carry):
        iv = idx_v[pl.ds(g * L, L)]
        tv = tgt_v[pl.ds(g * L, L)]
        fi_v[pl.ds(g * L, L)] = iv * VOCAB + tv
        return carry

    lax.fori_loop(0, PER_W // L, fi_step, 0)

    def pick_step(k, carry):
        pltpu.async_copy(
            tflat_hbm.at[fi_v.at[pl.ds(k * GRP, GRP)]],
            picked_v.at[pl.ds(k * GRP, GRP)], sem).wait()
        return carry

    lax.fori_loop(0, NGRP, pick_step, 0)

    def acc_step(g, acc):
        iv = idx_v[pl.ds(g * L, L)]
        lz = plsc.load_gather(logz_v, [iv])
        pk = picked_v[pl.ds(g * L, L)]
        return acc + (lz - pk)

    acc = lax.fori_loop(0, PER_W // L, acc_step, jnp.zeros((L,), jnp.float32))
    acc_v[...] = acc
    pltpu.sync_copy(acc_v, part_hbm.at[wid])


def _sc_loss(table_flat, idx_flat, tgt_flat, logz):
    mesh = plsc.VectorSubcoreMesh(core_axis_name="c", subcore_axis_name="s")
    f = pl.kernel(
        _loss_body,
        out_type=jax.ShapeDtypeStruct((NW, L), jnp.float32),
        mesh=mesh,
        compiler_params=pltpu.CompilerParams(
            needs_layout_passes=False, use_tc_tiling_on_sc=False),
        scratch_types=[
            pltpu.VMEM((PER_W,), jnp.int32),
            pltpu.VMEM((PER_W,), jnp.int32),
            pltpu.VMEM((PER_W,), jnp.int32),
            pltpu.VMEM((PER_W,), jnp.float32),
            pltpu.VMEM((VOCAB,), jnp.float32),
            pltpu.VMEM((L,), jnp.float32),
            pltpu.SemaphoreType.DMA,
        ],
    )
    return f(table_flat, idx_flat, tgt_flat, logz)


TRIM_R = 512


def _trim_body(x_ref, o_ref):
    o_ref[...] = x_ref[...].T[:VOCAB, :]


def _tc_trim(outp):
    return pl.pallas_call(
        _trim_body,
        grid=(BT // TRIM_R,),
        in_specs=[pl.BlockSpec((TRIM_R, VPAD), lambda i: (i, 0))],
        out_specs=pl.BlockSpec((VOCAB, TRIM_R), lambda i: (0, i)),
        out_shape=jax.ShapeDtypeStruct((VOCAB, BT), jnp.float32),
    )(outp)


def kernel(idx, targets, table):
    idx_flat = idx.reshape(-1).astype(jnp.int32)
    tgt_flat = targets.reshape(-1).astype(jnp.int32)
    logz = _row_logsumexp(table)
    logits2 = _sc_gather(table, idx_flat, VOCAB)
    partials = _sc_loss(table.reshape(-1), idx_flat, tgt_flat, logz)
    loss = jnp.sum(partials) / BT
    return (logits2, loss)


# speedup vs baseline: 1.6632x; 1.6632x over previous
import jax
import jax.numpy as jnp
from jax import lax
from jax.experimental import pallas as pl
from jax.experimental.pallas import tpu as pltpu
from jax.experimental.pallas import tpu_sc as plsc

VOCAB = 1000
VPAD = 1024
BT = 4096 * 50
NC, NS, L = 2, 16, 16
NW = NC * NS
PER_W = BT // NW
CHUNK = 32
NBUF = 2
P = 4
SLICE = BT // P
PW_G = SLICE // NW
NCHUNK_G = PW_G // CHUNK


def _logz_body(t_ref, o_ref):
    x = t_ref[...]
    m = jnp.max(x, axis=1)
    s = jnp.sum(jnp.exp(x - m[:, None]), axis=1)
    o_ref[...] = m + jnp.log(s)


def _row_logsumexp(table):
    return pl.pallas_call(
        _logz_body,
        out_shape=jax.ShapeDtypeStruct((VOCAB,), jnp.float32),
    )(table)


def _gather_body(tablep_hbm, idx_hbm, outp_hbm,
                 idx_v, rows0_v, rows1_v, sem0, sem1):
    wid = lax.axis_index("s") * NC + lax.axis_index("c")
    base = wid * PW_G
    rows_bufs = (rows0_v, rows1_v)
    sems = (sem0, sem1)

    pltpu.sync_copy(idx_hbm.at[pl.ds(base, PW_G)], idx_v)

    def gather(c, b):
        return pltpu.async_copy(
            tablep_hbm.at[idx_v.at[pl.ds(c * CHUNK, CHUNK)]],
            rows_bufs[b], sems[b])

    for b in range(NBUF):
        gather(b, b)

    def outer_step(g, carry):
        for b in range(NBUF):
            c = g * NBUF + b
            rows_v = rows_bufs[b]
            pltpu.make_async_copy(
                tablep_hbm.at[idx_v.at[pl.ds(c * CHUNK, CHUNK)]],
                rows_v, sems[b]).wait()
            pltpu.sync_copy(rows_v, outp_hbm.at[pl.ds(base + c * CHUNK, CHUNK)])

            @pl.when(c + NBUF < NCHUNK_G)
            def _():
                gather(c + NBUF, b)
        return carry

    lax.fori_loop(0, NCHUNK_G // NBUF, outer_step, 0)


def _sc_gather(tablep, idx_slice):
    mesh = plsc.VectorSubcoreMesh(core_axis_name="c", subcore_axis_name="s")
    f = pl.kernel(
        _gather_body,
        out_type=jax.ShapeDtypeStruct((SLICE, VPAD), jnp.float32),
        mesh=mesh,
        compiler_params=pltpu.CompilerParams(needs_layout_passes=False),
        scratch_types=[
            pltpu.VMEM((PW_G,), jnp.int32),
            pltpu.VMEM((CHUNK, VPAD), jnp.float32),
            pltpu.VMEM((CHUNK, VPAD), jnp.float32),
            pltpu.SemaphoreType.DMA,
            pltpu.SemaphoreType.DMA,
        ],
    )
    return f(tablep, idx_slice)


GRP = 128
NGRP = PER_W // GRP


def _loss_body(tflat_hbm, idx_hbm, tgt_hbm, logz_hbm, part_hbm,
               idx_v, tgt_v, fi_v, picked_v, logz_v, acc_v, sem):
    wid = lax.axis_index("s") * NC + lax.axis_index("c")
    base = wid * PER_W

    pltpu.sync_copy(idx_hbm.at[pl.ds(base, PER_W)], idx_v)
    pltpu.sync_copy(tgt_hbm.at[pl.ds(base, PER_W)], tgt_v)
    pltpu.sync_copy(logz_hbm, logz_v)

    def fi_step(g, carry):
        iv = idx_v[pl.ds(g * L, L)]
        tv = tgt_v[pl.ds(g * L, L)]
        fi_v[pl.ds(g * L, L)] = iv * VOCAB + tv
        return carry

    lax.fori_loop(0, PER_W // L, fi_step, 0)

    def pick_step(k, carry):
        pltpu.async_copy(
            tflat_hbm.at[fi_v.at[pl.ds(k * GRP, GRP)]],
            picked_v.at[pl.ds(k * GRP, GRP)], sem).wait()
        return carry

    lax.fori_loop(0, NGRP, pick_step, 0)

    def acc_step(g, acc):
        iv = idx_v[pl.ds(g * L, L)]
        lz = plsc.load_gather(logz_v, [iv])
        pk = picked_v[pl.ds(g * L, L)]
        return acc + (lz - pk)

    acc = lax.fori_loop(0, PER_W // L, acc_step, jnp.zeros((L,), jnp.float32))
    acc_v[...] = acc
    pltpu.sync_copy(acc_v, part_hbm.at[wid])


def _sc_loss(table_flat, idx_flat, tgt_flat, logz):
    mesh = plsc.VectorSubcoreMesh(core_axis_name="c", subcore_axis_name="s")
    f = pl.kernel(
        _loss_body,
        out_type=jax.ShapeDtypeStruct((NW, L), jnp.float32),
        mesh=mesh,
        compiler_params=pltpu.CompilerParams(
            needs_layout_passes=False, use_tc_tiling_on_sc=False),
        scratch_types=[
            pltpu.VMEM((PER_W,), jnp.int32),
            pltpu.VMEM((PER_W,), jnp.int32),
            pltpu.VMEM((PER_W,), jnp.int32),
            pltpu.VMEM((PER_W,), jnp.float32),
            pltpu.VMEM((VOCAB,), jnp.float32),
            pltpu.VMEM((L,), jnp.float32),
            pltpu.SemaphoreType.DMA,
        ],
    )
    return f(table_flat, idx_flat, tgt_flat, logz)


TRIM_R = 512


def _trim_body(x_ref, o_ref):
    o_ref[...] = x_ref[...].T[:VOCAB, :]


def _trim_body_acc(x_ref, f_ref, o_ref):
    del f_ref
    o_ref[...] = x_ref[...].T[:VOCAB, :]


NBLK = SLICE // TRIM_R


def _tc_trim_slice(outp_k, full_prev, k):
    out_spec = pl.BlockSpec((VOCAB, TRIM_R), lambda i, k=k: (0, k * NBLK + i))
    out_shape = jax.ShapeDtypeStruct((VOCAB, BT), jnp.float32)
    if full_prev is None:
        return pl.pallas_call(
            _trim_body,
            grid=(NBLK,),
            in_specs=[pl.BlockSpec((TRIM_R, VPAD), lambda i: (i, 0))],
            out_specs=out_spec,
            out_shape=out_shape,
        )(outp_k)
    return pl.pallas_call(
        _trim_body_acc,
        grid=(NBLK,),
        in_specs=[pl.BlockSpec((TRIM_R, VPAD), lambda i: (i, 0)),
                  pl.BlockSpec(memory_space=pl.ANY)],
        out_specs=out_spec,
        out_shape=out_shape,
        input_output_aliases={1: 0},
    )(outp_k, full_prev)


def kernel(idx, targets, table):
    idx_flat = idx.reshape(-1).astype(jnp.int32)
    tgt_flat = targets.reshape(-1).astype(jnp.int32)
    tablep = jnp.pad(table, ((0, 0), (0, VPAD - VOCAB)))
    logz = _row_logsumexp(table)
    full = None
    for k in range(P):
        outp_k = _sc_gather(tablep, idx_flat[k * SLICE:(k + 1) * SLICE])
        full = _tc_trim_slice(outp_k, full, k)
    logits2 = full.T
    partials = _sc_loss(table.reshape(-1), idx_flat, tgt_flat, logz)
    loss = jnp.sum(partials) / BT
    return (logits2, loss)
